# baseline (speedup 1.0000x reference)
import functools

import jax
import jax.numpy as jnp
from jax import lax
from jax.experimental import pallas as pl
from jax.experimental.pallas import tpu as pltpu
from jax.experimental.pallas import tpu_sc as plsc

KNB = 40
EPS = 1e-5
NEG = -3.0e38


def _knn_body(x_ref, xr_ref, idx_ref, *, rb):
    xb = x_ref[0]
    cdim, n = xb.shape
    xx = jnp.sum(xb * xb, axis=0, keepdims=True)
    xr = xr_ref[0]
    dot = lax.dot_general(xr, xb, (((0,), (0,)), ((), ())),
                          preferred_element_type=jnp.float32)
    q = 2.0 * dot - xx
    iota = lax.broadcasted_iota(jnp.int32, (rb, n), 1)
    tio = lax.broadcasted_iota(jnp.int32, (rb, KNB), 1)
    acc = jnp.zeros((rb, KNB), jnp.int32)
    for t in range(KNB):
        m = jnp.max(q, axis=1, keepdims=True)
        am = jnp.min(jnp.where(q == m, iota, n), axis=1, keepdims=True)
        acc = jnp.where(tio == t, am, acc)
        q = jnp.where(iota == am, NEG, q)
    idx_ref[0] = acc


def _knn(x):
    b, c, n = x.shape
    rb = 256
    return pl.pallas_call(
        functools.partial(_knn_body, rb=rb),
        grid=(b, n // rb),
        in_specs=[pl.BlockSpec((1, c, n), lambda i, r: (i, 0, 0)),
                  pl.BlockSpec((1, c, rb), lambda i, r: (i, 0, r))],
        out_specs=pl.BlockSpec((1, rb, KNB), lambda i, r: (i, r, 0)),
        out_shape=jax.ShapeDtypeStruct((b, n, KNB), jnp.int32),
    )(x, x)


def _scdiff(x, idx):
    b, c, n = x.shape
    p = 128
    pc = n // p
    nw = 32
    u_total = b * pc
    upw = u_total // nw
    assert upw * nw == u_total
    mesh = plsc.VectorSubcoreMesh(core_axis_name="c", subcore_axis_name="s",
                                  num_cores=2, num_subcores=16)

    @functools.partial(
        pl.kernel,
        out_type=jax.ShapeDtypeStruct((b, c, KNB, n), jnp.float32),
        mesh=mesh,
        compiler_params=pltpu.CompilerParams(needs_layout_passes=False),
        scratch_types=[
            pltpu.VMEM((c, n), jnp.float32),
            pltpu.VMEM((p, KNB), jnp.int32),
            pltpu.VMEM((8, KNB, p), jnp.float32),
        ],
    )
    def gk(x_h, idx_h, fd_h, table_v, idx_v, fd_v):
        wid = lax.axis_index("s") * 2 + lax.axis_index("c")
        for ui in range(upw):
            u = wid * upw + ui
            pci = lax.rem(u, pc)
            bi = u // pc
            n0 = pci * p
            if ui == 0:
                pltpu.sync_copy(x_h.at[bi], table_v)
            pltpu.sync_copy(idx_h.at[bi, pl.ds(n0, p), :], idx_v)

            def ccbody(cc8, carry):
                @plsc.parallel_loop(0, p // 16)
                def pgbody(pg, cc8=cc8):
                    pvec = lax.iota(jnp.int32, 16) + pg * 16
                    cvs = [jnp.full((16,), cc8 * 8 + i, jnp.int32)
                           for i in range(8)]
                    xns = [table_v[cc8 * 8 + i,
                                   pl.ds(n0 + pg * 16, 16)]
                           for i in range(8)]
                    for k in range(KNB):
                        iv = plsc.load_gather(
                            idx_v, [pvec, jnp.full((16,), k, jnp.int32)])
                        for i in range(8):
                            vj = plsc.load_gather(table_v, [cvs[i], iv])
                            fd_v[i, k, pl.ds(pg * 16, 16)] = vj - xns[i]
                pltpu.sync_copy(
                    fd_v, fd_h.at[bi, pl.ds(cc8 * 8, 8), :, pl.ds(n0, p)])
                return carry

            lax.fori_loop(0, c // 8, ccbody, 0)

    return gk(x, idx)


def _edge_body(fd_ref, x_ref, w_ref, mx_ref, mn_ref, sm_ref, sq_ref):
    _, c, k, p = fd_ref.shape
    co = w_ref.shape[0]
    fdm = fd_ref[0].reshape(c, k * p)
    xnb = jnp.broadcast_to(x_ref[0][:, None, :], (c, k, p)).reshape(c, k * p)
    f = jnp.concatenate([fdm, xnb], axis=0)
    h = lax.dot_general(w_ref[...], f, (((1,), (0,)), ((), ())),
                        preferred_element_type=jnp.float32)
    h3 = h.reshape(co, k, p)
    mx_ref[0] = jnp.max(h3, axis=1)
    mn_ref[0] = jnp.min(h3, axis=1)
    sm_ref[0] = jnp.sum(h3, axis=1)
    sq_ref[0] = jnp.sum(h3 * h3, axis=1)


def _edge(fd, x, w):
    b, c, k, n = fd.shape
    co = w.shape[0]
    p = 128
    os = pl.BlockSpec((1, co, p), lambda i, r: (i, 0, r))
    osh = jax.ShapeDtypeStruct((b, co, n), jnp.float32)
    return pl.pallas_call(
        _edge_body,
        grid=(b, n // p),
        in_specs=[
            pl.BlockSpec((1, c, k, p), lambda i, r: (i, 0, 0, r)),
            pl.BlockSpec((1, c, p), lambda i, r: (i, 0, r)),
            pl.BlockSpec((co, 2 * c), lambda i, r: (0, 0)),
        ],
        out_specs=[os, os, os, os],
        out_shape=[osh, osh, osh, osh],
    )(fd, x, w)


def _post_body(mx_ref, mn_ref, sm_ref, sq_ref, g_ref, b_ref, out_ref):
    _, c, n = mx_ref.shape
    cg = c // 2
    cnt = cg * n * float(KNB)
    stats = []
    for gi in range(2):
        sl = slice(gi * cg, (gi + 1) * cg)
        s1 = jnp.sum(sm_ref[0, sl, :])
        s2 = jnp.sum(sq_ref[0, sl, :])
        mean = s1 / cnt
        var = s2 / cnt - mean * mean
        stats.append((mean, jnp.sqrt(var + EPS)))
    ci = lax.broadcasted_iota(jnp.int32, (c, 1), 0)
    mean_c = jnp.where(ci < cg, stats[0][0], stats[1][0])
    std_c = jnp.where(ci < cg, stats[0][1], stats[1][1])
    gcol = g_ref[...]
    nmax = (mx_ref[0] - mean_c) / std_c
    nmin = (mn_ref[0] - mean_c) / std_c
    y = gcol * jnp.where(gcol >= 0, nmax, nmin) + b_ref[...]
    out_ref[0] = jnp.where(y >= 0, y, 0.2 * y)


def _post(mx, mn, sm, sq, g, bta):
    b, c, n = mx.shape
    fs = pl.BlockSpec((1, c, n), lambda i: (i, 0, 0))
    cs = pl.BlockSpec((c, 1), lambda i: (0, 0))
    return pl.pallas_call(
        _post_body,
        grid=(b,),
        in_specs=[fs, fs, fs, fs, cs, cs],
        out_specs=fs,
        out_shape=jax.ShapeDtypeStruct((b, c, n), jnp.float32),
    )(mx, mn, sm, sq, g, bta)


def _final_body(xf_ref, wm_ref, bm_ref, gm_ref, bt_ref, out_ref):
    co = wm_ref.shape[0]
    n = xf_ref.shape[2]
    h = lax.dot_general(wm_ref[...], xf_ref[0], (((1,), (0,)), ((), ())),
                        preferred_element_type=jnp.float32)
    h = h + bm_ref[...]
    hmax = jnp.max(h, axis=1, keepdims=True)
    hmin = jnp.min(h, axis=1, keepdims=True)
    hsum = jnp.sum(h, axis=1, keepdims=True)
    hsq = jnp.sum(h * h, axis=1, keepdims=True)
    cg = co // 8
    cnt = float(cg * n)
    means, stds = [], []
    for gi in range(8):
        sl = slice(gi * cg, (gi + 1) * cg)
        mean = jnp.sum(hsum[sl, :]) / cnt
        var = jnp.sum(hsq[sl, :]) / cnt - mean * mean
        means.append(mean)
        stds.append(jnp.sqrt(var + EPS))
    ci = lax.broadcasted_iota(jnp.int32, (co, 1), 0) // cg
    mean_c = jnp.zeros((co, 1), jnp.float32)
    std_c = jnp.zeros((co, 1), jnp.float32)
    for gi in range(8):
        mean_c = jnp.where(ci == gi, means[gi], mean_c)
        std_c = jnp.where(ci == gi, stds[gi], std_c)
    gcol = gm_ref[...]
    nmax = (hmax - mean_c) / std_c
    nmin = (hmin - mean_c) / std_c
    y = gcol * jnp.where(gcol >= 0, nmax, nmin) + bt_ref[...]
    out_ref[0] = jnp.maximum(y, 0.0)


def _final(xf, wm, bm, gm, bt):
    b, cin, n = xf.shape
    co = wm.shape[0]
    cs = pl.BlockSpec((co, 1), lambda i: (0, 0))
    out = pl.pallas_call(
        _final_body,
        grid=(b,),
        in_specs=[
            pl.BlockSpec((1, cin, n), lambda i: (i, 0, 0)),
            pl.BlockSpec((co, cin), lambda i: (0, 0)),
            cs, cs, cs,
        ],
        out_specs=pl.BlockSpec((1, co, 1), lambda i: (i, 0, 0)),
        out_shape=jax.ShapeDtypeStruct((b, co, 1), jnp.float32),
    )(xf, wm, bm, gm, bt)
    return out.reshape(b, co)


def _layer(x, idx, w, g, bta):
    fd = _scdiff(x, idx)
    mx, mn, sm, sq = _edge(fd, x, w)
    return _post(mx, mn, sm, sq, g.reshape(-1, 1), bta.reshape(-1, 1))


def kernel(x, W1, g1, b1, W2, g2, b2, W3, g3, b3, Wm, bm, gm, betam):
    b, c0, n = x.shape
    x0 = jnp.pad(x, ((0, 0), (0, 8 - c0), (0, 0)))
    w1 = jnp.concatenate(
        [jnp.pad(W1[:, :c0], ((0, 0), (0, 8 - c0))),
         jnp.pad(W1[:, c0:], ((0, 0), (0, 8 - c0)))], axis=1)
    idx1 = _knn(x0)
    x1 = _layer(x0, idx1, w1, g1, b1)
    idx2 = _knn(x1)
    x2 = _layer(x1, idx2, W2, g2, b2)
    x3 = _layer(x2, idx2, W3, g3, b3)
    xf = jnp.concatenate([x1, x2, x3], axis=1)
    x4 = _final(xf, Wm, bm.reshape(-1, 1), gm.reshape(-1, 1),
                betam.reshape(-1, 1))
    return x4, xf


# speedup vs baseline: 1.3970x; 1.3970x over previous
import functools

import jax
import jax.numpy as jnp
from jax import lax
from jax.experimental import pallas as pl
from jax.experimental.pallas import tpu as pltpu
from jax.experimental.pallas import tpu_sc as plsc

KNB = 40
EPS = 1e-5
NEG = -3.0e38


def _knn_body(xnc_ref, xcn_ref, xr_ref, idx_ref, *, rb):
    xcn = xcn_ref[0]
    n = xcn.shape[1]
    xx = jnp.sum(xcn * xcn, axis=0, keepdims=True)
    xr = xr_ref[0]
    dot = lax.dot_general(xr, xnc_ref[0], (((1,), (1,)), ((), ())),
                          preferred_element_type=jnp.float32)
    q = 2.0 * dot - xx
    iota = lax.broadcasted_iota(jnp.int32, (rb, n), 1).astype(jnp.float32)
    tio = lax.broadcasted_iota(jnp.int32, (rb, KNB), 1).astype(jnp.float32)
    nf = float(n)
    acc = jnp.zeros((rb, KNB), jnp.float32)
    for t in range(KNB):
        m = jnp.max(q, axis=1, keepdims=True)
        am = jnp.min(jnp.where(q == m, iota, nf), axis=1, keepdims=True)
        acc = jnp.where(tio == float(t), am, acc)
        q = jnp.where(iota == am, NEG, q)
    idx_ref[0] = acc.astype(jnp.int32)


def _knn(xnc, xcn):
    b, n, c = xnc.shape
    rb = 256
    return pl.pallas_call(
        functools.partial(_knn_body, rb=rb),
        grid=(b, n // rb),
        in_specs=[pl.BlockSpec((1, n, c), lambda i, r: (i, 0, 0)),
                  pl.BlockSpec((1, c, n), lambda i, r: (i, 0, 0)),
                  pl.BlockSpec((1, rb, c), lambda i, r: (i, r, 0))],
        out_specs=pl.BlockSpec((1, rb, KNB), lambda i, r: (i, r, 0)),
        out_shape=jax.ShapeDtypeStruct((b, n, KNB), jnp.int32),
    )(xnc, xcn, xnc)


def _scgather(xnc, idx_flat):
    b, n, c = xnc.shape
    p = 128
    pc = n // p
    nw = 32
    u_total = b * pc
    upw = u_total // nw
    assert upw * nw == u_total
    pg_rows = 16 * KNB
    mesh = plsc.VectorSubcoreMesh(core_axis_name="c", subcore_axis_name="s",
                                  num_cores=2, num_subcores=16)

    @functools.partial(
        pl.kernel,
        out_type=jax.ShapeDtypeStruct((b, n * KNB, c), jnp.float32),
        mesh=mesh,
        compiler_params=pltpu.CompilerParams(needs_layout_passes=False,
                                             use_tc_tiling_on_sc=False),
        scratch_types=[
            pltpu.VMEM((p // 16, pg_rows), jnp.int32),
            pltpu.VMEM((pg_rows, c), jnp.float32),
            pltpu.VMEM((pg_rows, c), jnp.float32),
            pltpu.SemaphoreType.DMA,
            pltpu.SemaphoreType.DMA,
        ],
    )
    def gk(x_h, idx_h, fg_h, idx_v, rows_a, rows_b, sem_a, sem_b):
        wid = lax.axis_index("s") * 2 + lax.axis_index("c")
        for ui in range(upw):
            u = wid * upw + ui
            pci = lax.rem(u, pc)
            bi = u // pc
            n0 = pci * p
            for pg in range(p // 16):
                pltpu.sync_copy(
                    idx_h.at[bi, pl.ds(n0 * KNB + pg * pg_rows, pg_rows)],
                    idx_v.at[pg])
            tab = x_h.at[bi]
            bufs = (rows_a, rows_b)
            sems = (sem_a, sem_b)
            cp = pltpu.async_copy(tab.at[idx_v.at[0]], bufs[0], sems[0])
            for pg in range(p // 16):
                if pg + 1 < p // 16:
                    cp_next = pltpu.async_copy(
                        tab.at[idx_v.at[pg + 1]],
                        bufs[(pg + 1) % 2], sems[(pg + 1) % 2])
                cp.wait()
                pltpu.sync_copy(
                    bufs[pg % 2],
                    fg_h.at[bi, pl.ds((n0 + pg * 16) * KNB, pg_rows), :])
                if pg + 1 < p // 16:
                    cp = cp_next

    return gk(xnc, idx_flat).reshape(b, n, KNB, c)


def _edge_body(fg_ref, x_ref, w_ref, mx_ref, mn_ref, sm_ref, sq_ref):
    _, p, k, c = fg_ref.shape
    fgm = fg_ref[0].reshape(p * k, c)
    xnb = jnp.broadcast_to(x_ref[0][:, None, :], (p, k, c)).reshape(p * k, c)
    f = jnp.concatenate([fgm - xnb, xnb], axis=1)
    h = lax.dot_general(f, w_ref[...], (((1,), (1,)), ((), ())),
                        preferred_element_type=jnp.float32)
    co = w_ref.shape[0]
    h3 = h.reshape(p, k, co)
    mx_ref[0] = jnp.max(h3, axis=1)
    mn_ref[0] = jnp.min(h3, axis=1)
    sm_ref[0] = jnp.sum(h3, axis=1)
    sq_ref[0] = jnp.sum(h3 * h3, axis=1)


def _edge(fg, xnc, w):
    b, n, k, c = fg.shape
    co = w.shape[0]
    p = 128
    os = pl.BlockSpec((1, p, co), lambda i, r: (i, r, 0))
    osh = jax.ShapeDtypeStruct((b, n, co), jnp.float32)
    return pl.pallas_call(
        _edge_body,
        grid=(b, n // p),
        in_specs=[
            pl.BlockSpec((1, p, k, c), lambda i, r: (i, r, 0, 0)),
            pl.BlockSpec((1, p, c), lambda i, r: (i, r, 0)),
            pl.BlockSpec((co, 2 * c), lambda i, r: (0, 0)),
        ],
        out_specs=[os, os, os, os],
        out_shape=[osh, osh, osh, osh],
    )(fg, xnc, w)


def _post_body(mx_ref, mn_ref, sm_ref, sq_ref, g_ref, b_ref,
               onc_ref, ocn_ref):
    _, n, c = mx_ref.shape
    cg = c // 2
    cnt = cg * n * float(KNB)
    stats = []
    for gi in range(2):
        sl = slice(gi * cg, (gi + 1) * cg)
        mean = jnp.sum(sm_ref[0, :, sl]) / cnt
        var = jnp.sum(sq_ref[0, :, sl]) / cnt - mean * mean
        stats.append((mean, jnp.sqrt(var + EPS)))
    ci = lax.broadcasted_iota(jnp.int32, (1, c), 1)
    mean_c = jnp.where(ci < cg, stats[0][0], stats[1][0])
    std_c = jnp.where(ci < cg, stats[0][1], stats[1][1])
    grow = g_ref[...]
    nmax = (mx_ref[0] - mean_c) / std_c
    nmin = (mn_ref[0] - mean_c) / std_c
    y = grow * jnp.where(grow >= 0, nmax, nmin) + b_ref[...]
    y = jnp.where(y >= 0, y, 0.2 * y)
    onc_ref[0] = y
    ocn_ref[0] = y.T


def _post(mx, mn, sm, sq, g, bta):
    b, n, c = mx.shape
    fs = pl.BlockSpec((1, n, c), lambda i: (i, 0, 0))
    cs = pl.BlockSpec((1, c), lambda i: (0, 0))
    return pl.pallas_call(
        _post_body,
        grid=(b,),
        in_specs=[fs, fs, fs, fs, cs, cs],
        out_specs=[fs, pl.BlockSpec((1, c, n), lambda i: (i, 0, 0))],
        out_shape=[jax.ShapeDtypeStruct((b, n, c), jnp.float32),
                   jax.ShapeDtypeStruct((b, c, n), jnp.float32)],
    )(mx, mn, sm, sq, g, bta)


def _final_body(xf_ref, wm_ref, bm_ref, gm_ref, bt_ref, out_ref):
    co = wm_ref.shape[0]
    n = xf_ref.shape[2]
    h = lax.dot_general(wm_ref[...], xf_ref[0], (((1,), (0,)), ((), ())),
                        preferred_element_type=jnp.float32)
    h = h + bm_ref[...]
    hmax = jnp.max(h, axis=1, keepdims=True)
    hmin = jnp.min(h, axis=1, keepdims=True)
    hsum = jnp.sum(h, axis=1, keepdims=True)
    hsq = jnp.sum(h * h, axis=1, keepdims=True)
    cg = co // 8
    cnt = float(cg * n)
    means, stds = [], []
    for gi in range(8):
        sl = slice(gi * cg, (gi + 1) * cg)
        mean = jnp.sum(hsum[sl, :]) / cnt
        var = jnp.sum(hsq[sl, :]) / cnt - mean * mean
        means.append(mean)
        stds.append(jnp.sqrt(var + EPS))
    ci = lax.broadcasted_iota(jnp.int32, (co, 1), 0) // cg
    mean_c = jnp.zeros((co, 1), jnp.float32)
    std_c = jnp.zeros((co, 1), jnp.float32)
    for gi in range(8):
        mean_c = jnp.where(ci == gi, means[gi], mean_c)
        std_c = jnp.where(ci == gi, stds[gi], std_c)
    gcol = gm_ref[...]
    nmax = (hmax - mean_c) / std_c
    nmin = (hmin - mean_c) / std_c
    y = gcol * jnp.where(gcol >= 0, nmax, nmin) + bt_ref[...]
    out_ref[0] = jnp.maximum(y, 0.0)


def _final(xf, wm, bm, gm, bt):
    b, cin, n = xf.shape
    co = wm.shape[0]
    cs = pl.BlockSpec((co, 1), lambda i: (0, 0))
    out = pl.pallas_call(
        _final_body,
        grid=(b,),
        in_specs=[
            pl.BlockSpec((1, cin, n), lambda i: (i, 0, 0)),
            pl.BlockSpec((co, cin), lambda i: (0, 0)),
            cs, cs, cs,
        ],
        out_specs=pl.BlockSpec((1, co, 1), lambda i: (i, 0, 0)),
        out_shape=jax.ShapeDtypeStruct((b, co, 1), jnp.float32),
    )(xf, wm, bm, gm, bt)
    return out.reshape(b, co)


def _layer(xnc, idx, w, g, bta):
    b = xnc.shape[0]
    fg = _scgather(xnc, idx.reshape(b, -1))
    mx, mn, sm, sq = _edge(fg, xnc, w)
    return _post(mx, mn, sm, sq, g.reshape(1, -1), bta.reshape(1, -1))


def kernel(x, W1, g1, b1, W2, g2, b2, W3, g3, b3, Wm, bm, gm, betam):
    b, c0, n = x.shape
    cp = 16
    x0cn = jnp.pad(x, ((0, 0), (0, cp - c0), (0, 0)))
    x0nc = jnp.transpose(x0cn, (0, 2, 1))
    w1 = jnp.concatenate(
        [jnp.pad(W1[:, :c0], ((0, 0), (0, cp - c0))),
         jnp.pad(W1[:, c0:], ((0, 0), (0, cp - c0)))], axis=1)
    idx1 = _knn(x0nc, x0cn)
    x1nc, x1cn = _layer(x0nc, idx1, w1, g1, b1)
    idx2 = _knn(x1nc, x1cn)
    x2nc, x2cn = _layer(x1nc, idx2, W2, g2, b2)
    _, x3cn = _layer(x2nc, idx2, W3, g3, b3)
    xf = jnp.concatenate([x1cn, x2cn, x3cn], axis=1)
    x4 = _final(xf, Wm, bm.reshape(-1, 1), gm.reshape(-1, 1),
                betam.reshape(-1, 1))
    return x4, xf
